# Initial kernel scaffold; baseline (speedup 1.0000x reference)
import jax
import jax.numpy as jnp
from jax.experimental import pallas as pl


def kernel(x, table):
    raise NotImplementedError("write your pallas kernel here")


# speedup vs baseline: 1.1080x; 1.1080x over previous
import functools

import jax
import jax.numpy as jnp
from jax import lax
from jax.experimental import pallas as pl
from jax.experimental.pallas import tpu as pltpu
from jax.experimental.pallas import tpu_sc as plsc

DIM = 32
B = 16384 * 50
NC = 2
NS = 16
NW = NC * NS
B_PER_W = B // NW
CH = 1600
NCHUNK = B_PER_W // CH


def _make_gather():
    mesh = plsc.VectorSubcoreMesh(core_axis_name="c", subcore_axis_name="s")

    @functools.partial(
        pl.kernel,
        mesh=mesh,
        out_type=jax.ShapeDtypeStruct((B, DIM), jnp.float32),
        scratch_types=[
            pltpu.VMEM((B_PER_W,), jnp.int32),
            pltpu.VMEM((CH, DIM), jnp.float32),
            pltpu.SemaphoreType.DMA,
        ],
        compiler_params=pltpu.CompilerParams(use_tc_tiling_on_sc=False),
    )
    def gather_kernel(x_hbm, table_hbm, out_hbm, idx_v, rows_v, sem):
        wid = lax.axis_index("s") * NC + lax.axis_index("c")
        base = pl.multiple_of(wid * B_PER_W, 8)
        pltpu.sync_copy(x_hbm.at[pl.ds(base, B_PER_W)], idx_v)

        def chunk(c, carry):
            off = pl.multiple_of(c * CH, 8)
            pltpu.async_copy(
                table_hbm.at[idx_v.at[pl.ds(off, CH)]], rows_v, sem
            ).wait()
            pltpu.sync_copy(rows_v, out_hbm.at[pl.ds(base + off, CH)])
            return carry

        lax.fori_loop(0, NCHUNK, chunk, 0)

    return gather_kernel


_gather = _make_gather()


@jax.jit
def kernel(x, table):
    out = _gather(x.reshape(-1).astype(jnp.int32), table)
    return out.reshape(x.shape + (DIM,))


# speedup vs baseline: 1.1098x; 1.0016x over previous
import functools

import jax
import jax.numpy as jnp
from jax import lax
from jax.experimental import pallas as pl
from jax.experimental.pallas import tpu as pltpu
from jax.experimental.pallas import tpu_sc as plsc

DIM = 32
B = 16384 * 50
NC = 2
NS = 16
NW = NC * NS
B_PER_W = B // NW
CH = 1600
NCHUNK = B_PER_W // CH


def _make_gather():
    mesh = plsc.VectorSubcoreMesh(core_axis_name="c", subcore_axis_name="s")

    @functools.partial(
        pl.kernel,
        mesh=mesh,
        out_type=jax.ShapeDtypeStruct((B, DIM), jnp.float32),
        scratch_types=[
            pltpu.VMEM((B_PER_W,), jnp.int32),
            pltpu.VMEM((CH, DIM), jnp.float32),
            pltpu.VMEM((CH, DIM), jnp.float32),
            pltpu.SemaphoreType.DMA,
            pltpu.SemaphoreType.DMA,
            pltpu.SemaphoreType.DMA,
            pltpu.SemaphoreType.DMA,
        ],
        compiler_params=pltpu.CompilerParams(use_tc_tiling_on_sc=False),
    )
    def gather_kernel(x_hbm, table_hbm, out_hbm, idx_v, rows0, rows1,
                      gsem0, gsem1, wsem0, wsem1):
        wid = lax.axis_index("s") * NC + lax.axis_index("c")
        base = pl.multiple_of(wid * B_PER_W, 8)
        pltpu.sync_copy(x_hbm.at[pl.ds(base, B_PER_W)], idx_v)

        bufs = (rows0, rows1)
        gsems = (gsem0, gsem1)
        wsems = (wsem0, wsem1)

        def gath(c, b):
            off = pl.multiple_of(c * CH, 8)
            return pltpu.make_async_copy(
                table_hbm.at[idx_v.at[pl.ds(off, CH)]], bufs[b], gsems[b]
            )

        def writ(c, b):
            off = pl.multiple_of(c * CH, 8)
            return pltpu.make_async_copy(
                bufs[b], out_hbm.at[pl.ds(base + off, CH)], wsems[b]
            )

        gath(0, 0).start()
        for c in range(NCHUNK):
            b = c % 2
            gath(c, b).wait()
            writ(c, b).start()
            if c + 1 < NCHUNK:
                nb = (c + 1) % 2
                if c >= 1:
                    writ(c - 1, nb).wait()
                gath(c + 1, nb).start()
        writ(NCHUNK - 2, (NCHUNK - 2) % 2).wait()
        writ(NCHUNK - 1, (NCHUNK - 1) % 2).wait()

    return gather_kernel


_gather = _make_gather()


@jax.jit
def kernel(x, table):
    out = _gather(x.reshape(-1).astype(jnp.int32), table)
    return out.reshape(x.shape + (DIM,))


# speedup vs baseline: 1.1108x; 1.0009x over previous
import functools

import jax
import jax.numpy as jnp
from jax import lax
from jax.experimental import pallas as pl
from jax.experimental.pallas import tpu as pltpu
from jax.experimental.pallas import tpu_sc as plsc

DIM = 32
B = 16384 * 50
NC = 2
NS = 16
NW = NC * NS
B_PER_W = B // NW
NSTREAM = 4
CH = 400
RCH = NSTREAM * CH
ROUNDS = B_PER_W // RCH


def _make_gather():
    mesh = plsc.VectorSubcoreMesh(core_axis_name="c", subcore_axis_name="s")

    row_buf = pltpu.VMEM((CH, DIM), jnp.float32)

    @functools.partial(
        pl.kernel,
        mesh=mesh,
        out_type=jax.ShapeDtypeStruct((B, DIM), jnp.float32),
        scratch_types=[
            pltpu.VMEM((B_PER_W,), jnp.int32),
            [[row_buf] * NSTREAM, [row_buf] * NSTREAM],
            pltpu.SemaphoreType.DMA,
            pltpu.SemaphoreType.DMA,
            pltpu.SemaphoreType.DMA,
            pltpu.SemaphoreType.DMA,
        ],
        compiler_params=pltpu.CompilerParams(use_tc_tiling_on_sc=False),
    )
    def gather_kernel(x_hbm, table_hbm, out_hbm, idx_v, banks,
                      gsem0, gsem1, wsem0, wsem1):
        wid = lax.axis_index("s") * NC + lax.axis_index("c")
        base = pl.multiple_of(wid * B_PER_W, 8)
        pltpu.sync_copy(x_hbm.at[pl.ds(base, B_PER_W)], idx_v)

        gsems = (gsem0, gsem1)
        wsems = (wsem0, wsem1)

        def gath(r, s, p):
            off = pl.multiple_of(r * RCH + s * CH, 8)
            return pltpu.make_async_copy(
                table_hbm.at[idx_v.at[pl.ds(off, CH)]], banks[p][s], gsems[p]
            )

        def writ(r, s, p):
            off = pl.multiple_of(r * RCH + s * CH, 8)
            return pltpu.make_async_copy(
                banks[p][s], out_hbm.at[pl.ds(base + off, CH)], wsems[p]
            )

        def body(i, carry):
            r0 = 2 * i
            r1 = 2 * i + 1

            @pl.when(i > 0)
            def _():
                for s in range(NSTREAM):
                    writ(0, s, 0).wait()

            for s in range(NSTREAM):
                gath(r0, s, 0).start()

            @pl.when(i > 0)
            def _():
                for s in range(NSTREAM):
                    writ(0, s, 1).wait()

            for s in range(NSTREAM):
                gath(r1, s, 1).start()
            for s in range(NSTREAM):
                gath(r0, s, 0).wait()
            for s in range(NSTREAM):
                writ(r0, s, 0).start()
            for s in range(NSTREAM):
                gath(r1, s, 1).wait()
            for s in range(NSTREAM):
                writ(r1, s, 1).start()
            return carry

        lax.fori_loop(0, ROUNDS // 2, body, 0)
        for p in range(2):
            for s in range(NSTREAM):
                writ(0, s, p).wait()

    return gather_kernel


_gather = _make_gather()


@jax.jit
def kernel(x, table):
    out = _gather(x.reshape(-1).astype(jnp.int32), table)
    return out.reshape(x.shape + (DIM,))
